# baseline (speedup 1.0000x reference)
import functools

import jax
import jax.numpy as jnp
from jax import lax
from jax.experimental import pallas as pl
from jax.experimental.pallas import tpu as pltpu
from jax.experimental.pallas import tpu_sc as plsc

_NUM_CORES = 2
_NUM_SUBCORES = 16
_INFLIGHT = 8


def _pool_sums(x, table, *, interpret=False):
    S, B = x.shape
    _, D = table.shape
    nw = _NUM_CORES * _NUM_SUBCORES
    bpw = B // nw
    k = _INFLIGHT
    assert B % nw == 0 and S % k == 0 and D % 16 == 0

    mesh = plsc.VectorSubcoreMesh(
        core_axis_name="c",
        subcore_axis_name="s",
        num_cores=_NUM_CORES,
        num_subcores=_NUM_SUBCORES,
    )

    @functools.partial(
        pl.kernel,
        out_type=jax.ShapeDtypeStruct((B, D), jnp.float32),
        mesh=mesh,
        interpret=interpret,
        compiler_params=pltpu.CompilerParams(use_tc_tiling_on_sc=False),
        scratch_types=[
            pltpu.VMEM((S, bpw), jnp.int32),
            pltpu.VMEM((bpw, D), jnp.float32),
            pltpu.SemaphoreType.DMA,
        ],
    )
    def pool(x_hbm, tab_hbm, out_hbm, xv, acc, sem):
        wid = lax.axis_index("s") * _NUM_CORES + lax.axis_index("c")
        base = wid * bpw
        pltpu.sync_copy(x_hbm.at[:, pl.ds(base, bpw)], xv)

        zv = jnp.zeros((16,), jnp.float32)

        def zrow(i, _):
            def zcol(j, _2):
                acc[i, pl.ds(j * 16, 16)] = zv
                return 0

            return lax.fori_loop(0, D // 16, zcol, 0, unroll=True)

        lax.fori_loop(0, bpw, zrow, 0)

        def chunk(c, _):
            s0 = c * k
            for j in range(k):
                pltpu.async_copy(tab_hbm.at[xv.at[s0 + j]], acc, sem, add=True)
            for _j in range(k):
                pltpu.make_async_copy(tab_hbm.at[xv.at[s0]], acc, sem).wait()
            return 0

        lax.fori_loop(0, S // k, chunk, 0)
        pltpu.sync_copy(acc, out_hbm.at[pl.ds(base, bpw)])

    return pool(x, table)


def _fc(sums, w, b2, inv_s, *, interpret=False):
    B, _ = sums.shape
    C = w.shape[1]

    def body(s_ref, w_ref, b_ref, o_ref):
        o_ref[...] = (
            jnp.dot(s_ref[...], w_ref[...], preferred_element_type=jnp.float32)
            * inv_s
            + b_ref[...]
        )

    return pl.pallas_call(
        body,
        out_shape=jax.ShapeDtypeStruct((B, C), jnp.float32),
        interpret=interpret,
    )(sums, w, b2)


def kernel(x, table, W, b):
    sums = _pool_sums(x, table)
    return _fc(sums, W, b.reshape(1, -1), 1.0 / x.shape[0])
